# baseline (speedup 1.0000x reference)
import functools

import jax
import jax.numpy as jnp
from jax import lax
from jax.experimental import pallas as pl
from jax.experimental.pallas import tpu as pltpu
from jax.experimental.pallas import tpu_sc as plsc

VOCAB = 1000000
DIM = 64
BATCH = 4096
SEQ = 50
PAD_IDX = 0

L = 16
NW = 32
B_PER_W = BATCH // NW
SEQ_PAD = 128
ROWS_PER_CHUNK = 2
CHUNK_IDX = ROWS_PER_CHUNK * SEQ
N_CHUNKS = B_PER_W // ROWS_PER_CHUNK


DIM_PAD = 128


def _encoder_kernel(tokens_hbm, table_hbm, out_hbm,
                    idxp_v, idx_v, rows0_v, rows1_v, rows2_v, rows3_v,
                    row0_v, out_v, sem0, sem1, sem2, sem3):
    cid = lax.axis_index("c")
    sid = lax.axis_index("s")
    wid = sid * 2 + cid

    pltpu.sync_copy(tokens_hbm.at[pl.ds(wid * B_PER_W, B_PER_W)], idxp_v)
    pltpu.sync_copy(table_hbm.at[pl.ds(0, 1)], row0_v)

    for r in range(B_PER_W):
        c, half = r // 2, (r % 2) * SEQ
        for off in (0, L, 2 * L, 34):
            idx_v[c, pl.ds(half + off, L)] = idxp_v[r, pl.ds(off, L)]

    one = jnp.ones((L,), jnp.int32)
    izero = jnp.zeros((L,), jnp.int32)
    zeros = jnp.zeros((L,), jnp.float32)
    row0 = [row0_v[0, pl.ds(k * L, L)] for k in range(4)]

    def count_nonpad(row):
        cnt = izero
        for off in (0, L, 2 * L, 3 * L):
            toks = idxp_v[row, pl.ds(off, L)]
            cnt = cnt + jnp.where(toks != PAD_IDX, one, izero)
        parts = [cnt[i] for i in range(L)]
        while len(parts) > 1:
            parts = [parts[i] + parts[i + 1] for i in range(0, len(parts), 2)]
        return parts[0]

    def compute_chunk(c, rows_v):
        for r in range(ROWS_PER_CHUNK):
            n1 = jnp.full((L,), count_nonpad(2 * c + r), jnp.float32)
            recip = 1.0 / jnp.maximum(n1, 1.0)
            n0f = (SEQ - n1) * recip
            acc = [zeros, zeros, zeros, zeros]
            for t in range(SEQ):
                slot = r * SEQ + t
                for k in range(4):
                    acc[k] = acc[k] + rows_v[slot, pl.ds(k * L, L)]
            orow = 2 * c + r
            for k in range(4):
                out_v[orow, pl.ds(k * L, L)] = acc[k] * recip - n0f * row0[k]

    bufs = (rows0_v, rows1_v, rows2_v, rows3_v)
    sems = (sem0, sem1, sem2, sem3)
    nbuf = 4

    def gather(c, b):
        return pltpu.async_copy(table_hbm.at[idx_v.at[c]], bufs[b], sems[b])

    def wait(c, b):
        pltpu.make_async_copy(table_hbm.at[idx_v.at[c]], bufs[b],
                              sems[b]).wait()

    for b in range(nbuf - 1):
        gather(b, b)

    def body(i, carry):
        c0 = nbuf * i
        for j in range(nbuf):
            c = c0 + j
            nxt = c + nbuf - 1
            nxt_b = (j + nbuf - 1) % nbuf

            @pl.when(nxt < N_CHUNKS)
            def _():
                gather(nxt, nxt_b)
            wait(c, j)
            compute_chunk(c, bufs[j])
        return carry

    lax.fori_loop(0, N_CHUNKS // nbuf, body, 0)

    pltpu.sync_copy(out_v, out_hbm.at[pl.ds(wid * B_PER_W, B_PER_W)])


TP_BLK = 512


def _transpose_pad_kernel(in_ref, out_ref):
    t = in_ref[...].T
    out_ref[:, :DIM] = t
    out_ref[:, DIM:] = jnp.zeros((TP_BLK, DIM_PAD - DIM), jnp.float32)


def _transpose_pad(table_t):
    grid = (VOCAB + TP_BLK - 1) // TP_BLK
    return pl.pallas_call(
        _transpose_pad_kernel,
        grid=(grid,),
        in_specs=[pl.BlockSpec((DIM, TP_BLK), lambda i: (0, i))],
        out_specs=pl.BlockSpec((TP_BLK, DIM_PAD), lambda i: (i, 0)),
        out_shape=jax.ShapeDtypeStruct((VOCAB, DIM_PAD), jnp.float32),
    )(table_t)


@jax.jit
def kernel(token_ids, table):
    tokens_pad = jnp.pad(token_ids, ((0, 0), (0, SEQ_PAD - SEQ)))
    table_pad = _transpose_pad(table.T)
    mesh = plsc.VectorSubcoreMesh(core_axis_name="c", subcore_axis_name="s")
    f = functools.partial(
        pl.kernel,
        mesh=mesh,
        compiler_params=pltpu.CompilerParams(use_tc_tiling_on_sc=True),
        out_type=jax.ShapeDtypeStruct((BATCH, DIM), jnp.float32),
        scratch_types=[
            pltpu.VMEM((B_PER_W, SEQ_PAD), jnp.int32),
            pltpu.VMEM((N_CHUNKS, CHUNK_IDX), jnp.int32),
            pltpu.VMEM((CHUNK_IDX, DIM_PAD), jnp.float32),
            pltpu.VMEM((CHUNK_IDX, DIM_PAD), jnp.float32),
            pltpu.VMEM((CHUNK_IDX, DIM_PAD), jnp.float32),
            pltpu.VMEM((CHUNK_IDX, DIM_PAD), jnp.float32),
            pltpu.VMEM((1, DIM_PAD), jnp.float32),
            pltpu.VMEM((B_PER_W, DIM), jnp.float32),
            pltpu.SemaphoreType.DMA,
            pltpu.SemaphoreType.DMA,
            pltpu.SemaphoreType.DMA,
            pltpu.SemaphoreType.DMA,
        ],
    )(_encoder_kernel)
    return f(tokens_pad, table_pad)


# speedup vs baseline: 3.6970x; 3.6970x over previous
import functools

import jax
import jax.numpy as jnp
from jax import lax
from jax.experimental import pallas as pl
from jax.experimental.pallas import tpu as pltpu
from jax.experimental.pallas import tpu_sc as plsc

VOCAB = 1000000
DIM = 64
BATCH = 4096
SEQ = 50
PAD_IDX = 0

L = 16
NW = 32
B_PER_W = BATCH // NW
SEQ_PAD = 128
ROWS_PER_CHUNK = 2
CHUNK_IDX = ROWS_PER_CHUNK * SEQ
N_CHUNKS = B_PER_W // ROWS_PER_CHUNK


DIM_PAD = 128


def _encoder_kernel(tokens_hbm, table_hbm, out_hbm,
                    idxp_v, idx_v, rows0_v, rows1_v, rows2_v, rows3_v,
                    row0_v, out_v, sem0, sem1, sem2, sem3):
    cid = lax.axis_index("c")
    sid = lax.axis_index("s")
    wid = sid * 2 + cid

    pltpu.sync_copy(tokens_hbm.at[pl.ds(wid * B_PER_W, B_PER_W)], idxp_v)
    pltpu.sync_copy(table_hbm.at[pl.ds(0, 1)], row0_v)

    for r in range(B_PER_W):
        c, half = r // 2, (r % 2) * SEQ
        for off in (0, L, 2 * L, 34):
            idx_v[c, pl.ds(half + off, L)] = idxp_v[r, pl.ds(off, L)]

    one = jnp.ones((L,), jnp.int32)
    izero = jnp.zeros((L,), jnp.int32)
    zeros = jnp.zeros((L,), jnp.float32)
    row0 = [row0_v[0, pl.ds(k * L, L)] for k in range(4)]

    def count_nonpad(row):
        cnt = izero
        for off in (0, L, 2 * L, 3 * L):
            toks = idxp_v[row, pl.ds(off, L)]
            cnt = cnt + jnp.where(toks != PAD_IDX, one, izero)
        parts = [cnt[i] for i in range(L)]
        while len(parts) > 1:
            parts = [parts[i] + parts[i + 1] for i in range(0, len(parts), 2)]
        return parts[0]

    def compute_chunk(c, rows_v):
        for r in range(ROWS_PER_CHUNK):
            n1 = jnp.full((L,), count_nonpad(2 * c + r), jnp.float32)
            recip = 1.0 / jnp.maximum(n1, 1.0)
            n0f = (SEQ - n1) * recip
            acc = [zeros, zeros, zeros, zeros]
            for t in range(SEQ):
                slot = r * SEQ + t
                for k in range(4):
                    acc[k] = acc[k] + rows_v[slot, pl.ds(k * L, L)]
            orow = 2 * c + r
            for k in range(4):
                out_v[orow, pl.ds(k * L, L)] = acc[k] * recip - n0f * row0[k]

    bufs = (rows0_v, rows1_v, rows2_v, rows3_v)
    sems = (sem0, sem1, sem2, sem3)
    nbuf = 4

    def gather(c, b):
        return pltpu.async_copy(table_hbm.at[idx_v.at[c]], bufs[b], sems[b])

    def wait(c, b):
        pltpu.make_async_copy(table_hbm.at[idx_v.at[c]], bufs[b],
                              sems[b]).wait()

    for b in range(nbuf - 1):
        gather(b, b)

    def body(i, carry):
        c0 = nbuf * i
        for j in range(nbuf):
            c = c0 + j
            nxt = c + nbuf - 1
            nxt_b = (j + nbuf - 1) % nbuf

            @pl.when(nxt < N_CHUNKS)
            def _():
                gather(nxt, nxt_b)
            wait(c, j)
            compute_chunk(c, bufs[j])
        return carry

    lax.fori_loop(0, N_CHUNKS // nbuf, body, 0)

    pltpu.sync_copy(out_v, out_hbm.at[pl.ds(wid * B_PER_W, B_PER_W)])


TP_BLK = 8192


def _transpose_pad_kernel(in_ref, out_ref):
    t = in_ref[...].T
    out_ref[:, :DIM] = t
    out_ref[:, DIM:] = jnp.zeros((TP_BLK, DIM_PAD - DIM), jnp.float32)


def _transpose_pad(table_t):
    grid = (VOCAB + TP_BLK - 1) // TP_BLK
    return pl.pallas_call(
        _transpose_pad_kernel,
        grid=(grid,),
        in_specs=[pl.BlockSpec((DIM, TP_BLK), lambda i: (0, i))],
        out_specs=pl.BlockSpec((TP_BLK, DIM_PAD), lambda i: (i, 0)),
        out_shape=jax.ShapeDtypeStruct((VOCAB, DIM_PAD), jnp.float32),
    )(table_t)


@jax.jit
def kernel(token_ids, table):
    tokens_pad = jnp.pad(token_ids, ((0, 0), (0, SEQ_PAD - SEQ)))
    table_pad = _transpose_pad(table.T)
    mesh = plsc.VectorSubcoreMesh(core_axis_name="c", subcore_axis_name="s")
    f = functools.partial(
        pl.kernel,
        mesh=mesh,
        compiler_params=pltpu.CompilerParams(use_tc_tiling_on_sc=True),
        out_type=jax.ShapeDtypeStruct((BATCH, DIM), jnp.float32),
        scratch_types=[
            pltpu.VMEM((B_PER_W, SEQ_PAD), jnp.int32),
            pltpu.VMEM((N_CHUNKS, CHUNK_IDX), jnp.int32),
            pltpu.VMEM((CHUNK_IDX, DIM_PAD), jnp.float32),
            pltpu.VMEM((CHUNK_IDX, DIM_PAD), jnp.float32),
            pltpu.VMEM((CHUNK_IDX, DIM_PAD), jnp.float32),
            pltpu.VMEM((CHUNK_IDX, DIM_PAD), jnp.float32),
            pltpu.VMEM((1, DIM_PAD), jnp.float32),
            pltpu.VMEM((B_PER_W, DIM), jnp.float32),
            pltpu.SemaphoreType.DMA,
            pltpu.SemaphoreType.DMA,
            pltpu.SemaphoreType.DMA,
            pltpu.SemaphoreType.DMA,
        ],
    )(_encoder_kernel)
    return f(tokens_pad, table_pad)


# speedup vs baseline: 3.9282x; 1.0625x over previous
import functools

import jax
import jax.numpy as jnp
from jax import lax
from jax.experimental import pallas as pl
from jax.experimental.pallas import tpu as pltpu
from jax.experimental.pallas import tpu_sc as plsc

VOCAB = 1000000
DIM = 64
BATCH = 4096
SEQ = 50
PAD_IDX = 0

L = 16
NW = 32
B_PER_W = BATCH // NW
SEQ_PAD = 128
ROWS_PER_CHUNK = 2
CHUNK_IDX = ROWS_PER_CHUNK * SEQ
N_CHUNKS = B_PER_W // ROWS_PER_CHUNK


DIM_PAD = 128


def _encoder_kernel(tokens_hbm, table_hbm, out_hbm,
                    idxp_v, idx_v, rows0_v, rows1_v, rows2_v, rows3_v,
                    row0_v, out_v, sem0, sem1, sem2, sem3):
    cid = lax.axis_index("c")
    sid = lax.axis_index("s")
    wid = sid * 2 + cid

    pltpu.sync_copy(tokens_hbm.at[pl.ds(wid * B_PER_W, B_PER_W)], idxp_v)
    pltpu.sync_copy(table_hbm.at[pl.ds(0, 1)], row0_v)

    for r in range(B_PER_W):
        c, half = r // 2, (r % 2) * SEQ
        for off in (0, L, 2 * L, 34):
            idx_v[c, pl.ds(half + off, L)] = idxp_v[r, pl.ds(off, L)]

    one = jnp.ones((L,), jnp.int32)
    izero = jnp.zeros((L,), jnp.int32)
    zeros = jnp.zeros((L,), jnp.float32)
    row0 = [row0_v[0, pl.ds(k * L, L)] for k in range(4)]

    def count_nonpad(row):
        cnt = izero
        for off in (0, L, 2 * L, 3 * L):
            toks = idxp_v[row, pl.ds(off, L)]
            cnt = cnt + jnp.where(toks != PAD_IDX, one, izero)
        parts = [cnt[i] for i in range(L)]
        while len(parts) > 1:
            parts = [parts[i] + parts[i + 1] for i in range(0, len(parts), 2)]
        return parts[0]

    def compute_chunk(c, rows_v):
        for r in range(ROWS_PER_CHUNK):
            n1 = jnp.full((L,), count_nonpad(2 * c + r), jnp.float32)
            recip = 1.0 / jnp.maximum(n1, 1.0)
            n0f = (SEQ - n1) * recip
            acc = [zeros, zeros, zeros, zeros]
            for t in range(SEQ):
                slot = r * SEQ + t
                for k in range(4):
                    acc[k] = acc[k] + rows_v[slot, pl.ds(k * L, L)]
            orow = 2 * c + r
            for k in range(4):
                out_v[orow, pl.ds(k * L, L)] = acc[k] * recip - n0f * row0[k]

    bufs = (rows0_v, rows1_v, rows2_v, rows3_v)
    sems = (sem0, sem1, sem2, sem3)
    nbuf = 4

    def gather(c, b):
        return pltpu.async_copy(table_hbm.at[idx_v.at[c]], bufs[b], sems[b])

    def wait(c, b):
        pltpu.make_async_copy(table_hbm.at[idx_v.at[c]], bufs[b],
                              sems[b]).wait()

    for b in range(nbuf - 1):
        gather(b, b)

    def body(i, carry):
        c0 = nbuf * i
        for j in range(nbuf):
            c = c0 + j
            nxt = c + nbuf - 1
            nxt_b = (j + nbuf - 1) % nbuf

            @pl.when(nxt < N_CHUNKS)
            def _():
                gather(nxt, nxt_b)
            wait(c, j)
            compute_chunk(c, bufs[j])
        return carry

    lax.fori_loop(0, N_CHUNKS // nbuf, body, 0)

    pltpu.sync_copy(out_v, out_hbm.at[pl.ds(wid * B_PER_W, B_PER_W)])


TP_BLK = 16384


def _transpose_pad_kernel(in_ref, out_ref):
    t = in_ref[...].T
    out_ref[:, :DIM] = t
    out_ref[:, DIM:] = jnp.zeros((TP_BLK, DIM_PAD - DIM), jnp.float32)


def _transpose_pad(table_t):
    grid = (VOCAB + TP_BLK - 1) // TP_BLK
    return pl.pallas_call(
        _transpose_pad_kernel,
        grid=(grid,),
        in_specs=[pl.BlockSpec((DIM, TP_BLK), lambda i: (0, i))],
        out_specs=pl.BlockSpec((TP_BLK, DIM_PAD), lambda i: (i, 0)),
        out_shape=jax.ShapeDtypeStruct((VOCAB, DIM_PAD), jnp.float32),
    )(table_t)


@jax.jit
def kernel(token_ids, table):
    tokens_pad = jnp.pad(token_ids, ((0, 0), (0, SEQ_PAD - SEQ)))
    table_pad = _transpose_pad(table.T)
    mesh = plsc.VectorSubcoreMesh(core_axis_name="c", subcore_axis_name="s")
    f = functools.partial(
        pl.kernel,
        mesh=mesh,
        compiler_params=pltpu.CompilerParams(use_tc_tiling_on_sc=True),
        out_type=jax.ShapeDtypeStruct((BATCH, DIM), jnp.float32),
        scratch_types=[
            pltpu.VMEM((B_PER_W, SEQ_PAD), jnp.int32),
            pltpu.VMEM((N_CHUNKS, CHUNK_IDX), jnp.int32),
            pltpu.VMEM((CHUNK_IDX, DIM_PAD), jnp.float32),
            pltpu.VMEM((CHUNK_IDX, DIM_PAD), jnp.float32),
            pltpu.VMEM((CHUNK_IDX, DIM_PAD), jnp.float32),
            pltpu.VMEM((CHUNK_IDX, DIM_PAD), jnp.float32),
            pltpu.VMEM((1, DIM_PAD), jnp.float32),
            pltpu.VMEM((B_PER_W, DIM), jnp.float32),
            pltpu.SemaphoreType.DMA,
            pltpu.SemaphoreType.DMA,
            pltpu.SemaphoreType.DMA,
            pltpu.SemaphoreType.DMA,
        ],
    )(_encoder_kernel)
    return f(tokens_pad, table_pad)


# speedup vs baseline: 4.0013x; 1.0186x over previous
import functools

import jax
import jax.numpy as jnp
from jax import lax
from jax.experimental import pallas as pl
from jax.experimental.pallas import tpu as pltpu
from jax.experimental.pallas import tpu_sc as plsc

VOCAB = 1000000
DIM = 64
BATCH = 4096
SEQ = 50
PAD_IDX = 0

L = 16
NW = 32
B_PER_W = BATCH // NW
SEQ_PAD = 128
ROWS_PER_CHUNK = 2
CHUNK_IDX = ROWS_PER_CHUNK * SEQ
N_CHUNKS = B_PER_W // ROWS_PER_CHUNK


DIM_PAD = 128


def _encoder_kernel(tokens_hbm, table_hbm, out_hbm,
                    idxp_v, idx_v, rows0_v, rows1_v, rows2_v, rows3_v,
                    row0_v, out_v, sem0, sem1, sem2, sem3):
    cid = lax.axis_index("c")
    sid = lax.axis_index("s")
    wid = sid * 2 + cid

    pltpu.sync_copy(tokens_hbm.at[pl.ds(wid * B_PER_W, B_PER_W)], idxp_v)
    pltpu.sync_copy(table_hbm.at[pl.ds(0, 1)], row0_v)

    for r in range(B_PER_W):
        c, half = r // 2, (r % 2) * SEQ
        for off in (0, L, 2 * L, 34):
            idx_v[c, pl.ds(half + off, L)] = idxp_v[r, pl.ds(off, L)]

    one = jnp.ones((L,), jnp.int32)
    izero = jnp.zeros((L,), jnp.int32)
    zeros = jnp.zeros((L,), jnp.float32)
    row0 = [row0_v[0, pl.ds(k * L, L)] for k in range(4)]

    def count_nonpad(row):
        cnt = izero
        for off in (0, L, 2 * L, 3 * L):
            toks = idxp_v[row, pl.ds(off, L)]
            cnt = cnt + jnp.where(toks != PAD_IDX, one, izero)
        parts = [cnt[i] for i in range(L)]
        while len(parts) > 1:
            parts = [parts[i] + parts[i + 1] for i in range(0, len(parts), 2)]
        return parts[0]

    def compute_chunk(c, rows_v):
        for r in range(ROWS_PER_CHUNK):
            n1 = jnp.full((L,), count_nonpad(2 * c + r), jnp.float32)
            recip = 1.0 / jnp.maximum(n1, 1.0)
            n0f = (SEQ - n1) * recip
            acc = [zeros, zeros, zeros, zeros]
            for t in range(SEQ):
                slot = r * SEQ + t
                for k in range(4):
                    acc[k] = acc[k] + rows_v[slot, pl.ds(k * L, L)]
            orow = 2 * c + r
            for k in range(4):
                out_v[orow, pl.ds(k * L, L)] = acc[k] * recip - n0f * row0[k]

    bufs = (rows0_v, rows1_v, rows2_v, rows3_v)
    sems = (sem0, sem1, sem2, sem3)
    nbuf = 4

    def gather(c, b):
        return pltpu.async_copy(table_hbm.at[idx_v.at[c]], bufs[b], sems[b])

    def wait(c, b):
        pltpu.make_async_copy(table_hbm.at[idx_v.at[c]], bufs[b],
                              sems[b]).wait()

    for b in range(nbuf - 1):
        gather(b, b)

    def body(i, carry):
        c0 = nbuf * i
        for j in range(nbuf):
            c = c0 + j
            nxt = c + nbuf - 1
            nxt_b = (j + nbuf - 1) % nbuf

            @pl.when(nxt < N_CHUNKS)
            def _():
                gather(nxt, nxt_b)
            wait(c, j)
            compute_chunk(c, bufs[j])
        return carry

    lax.fori_loop(0, N_CHUNKS // nbuf, body, 0)

    pltpu.sync_copy(out_v, out_hbm.at[pl.ds(wid * B_PER_W, B_PER_W)])


TP_BLK = 32768


def _transpose_pad_kernel(in_ref, out_ref):
    t = in_ref[...].T
    out_ref[:, :DIM] = t
    out_ref[:, DIM:] = jnp.zeros((TP_BLK, DIM_PAD - DIM), jnp.float32)


def _transpose_pad(table_t):
    grid = (VOCAB + TP_BLK - 1) // TP_BLK
    return pl.pallas_call(
        _transpose_pad_kernel,
        grid=(grid,),
        in_specs=[pl.BlockSpec((DIM, TP_BLK), lambda i: (0, i))],
        out_specs=pl.BlockSpec((TP_BLK, DIM_PAD), lambda i: (i, 0)),
        out_shape=jax.ShapeDtypeStruct((VOCAB, DIM_PAD), jnp.float32),
    )(table_t)


@jax.jit
def kernel(token_ids, table):
    tokens_pad = jnp.pad(token_ids, ((0, 0), (0, SEQ_PAD - SEQ)))
    table_pad = _transpose_pad(table.T)
    mesh = plsc.VectorSubcoreMesh(core_axis_name="c", subcore_axis_name="s")
    f = functools.partial(
        pl.kernel,
        mesh=mesh,
        compiler_params=pltpu.CompilerParams(use_tc_tiling_on_sc=True),
        out_type=jax.ShapeDtypeStruct((BATCH, DIM), jnp.float32),
        scratch_types=[
            pltpu.VMEM((B_PER_W, SEQ_PAD), jnp.int32),
            pltpu.VMEM((N_CHUNKS, CHUNK_IDX), jnp.int32),
            pltpu.VMEM((CHUNK_IDX, DIM_PAD), jnp.float32),
            pltpu.VMEM((CHUNK_IDX, DIM_PAD), jnp.float32),
            pltpu.VMEM((CHUNK_IDX, DIM_PAD), jnp.float32),
            pltpu.VMEM((CHUNK_IDX, DIM_PAD), jnp.float32),
            pltpu.VMEM((1, DIM_PAD), jnp.float32),
            pltpu.VMEM((B_PER_W, DIM), jnp.float32),
            pltpu.SemaphoreType.DMA,
            pltpu.SemaphoreType.DMA,
            pltpu.SemaphoreType.DMA,
            pltpu.SemaphoreType.DMA,
        ],
    )(_encoder_kernel)
    return f(tokens_pad, table_pad)
